# baseline (device time: 32366 ns/iter reference)
import functools

import jax
import jax.numpy as jnp
from jax import lax
from jax.experimental import pallas as pl
from jax.experimental.pallas import tpu as pltpu

B, SQ, H, D = 2, 256, 8, 64
SCALE = D ** -0.5


def kernel(Q, K, V):
    def body(q_ref, k_ref, v_ref, o_ref, kbuf, vbuf, send_sems, recv_sems):
        my_x = lax.axis_index("x")
        my_y = lax.axis_index("y")
        nbr = (my_x, 1 - my_y)

        barrier = pltpu.get_barrier_semaphore()
        pl.semaphore_signal(
            barrier, inc=1, device_id=nbr, device_id_type=pl.DeviceIdType.MESH
        )
        pl.semaphore_wait(barrier, 1)

        kbuf[0] = k_ref[...].astype(jnp.bfloat16)
        vbuf[0] = v_ref[...].astype(jnp.bfloat16)

        rdma_k = pltpu.make_async_remote_copy(
            src_ref=kbuf.at[0],
            dst_ref=kbuf.at[1],
            send_sem=send_sems.at[0],
            recv_sem=recv_sems.at[0],
            device_id=nbr,
            device_id_type=pl.DeviceIdType.MESH,
        )
        rdma_v = pltpu.make_async_remote_copy(
            src_ref=vbuf.at[0],
            dst_ref=vbuf.at[1],
            send_sem=send_sems.at[1],
            recv_sem=recv_sems.at[1],
            device_id=nbr,
            device_id_type=pl.DeviceIdType.MESH,
        )
        rdma_k.start()
        rdma_v.start()
        rdma_k.wait()
        rdma_v.wait()
        if True:
            o_ref[...] = kbuf[1].astype(jnp.float32) + vbuf[1].astype(jnp.float32)
            return

        qv = (q_ref[...] * SCALE).astype(jnp.bfloat16)
        for b in range(B):
            for h in range(H):
                q = qv[b, :, h, :]
                k = jnp.concatenate([kbuf[0, b, :, h, :], kbuf[1, b, :, h, :]])
                v = jnp.concatenate([vbuf[0, b, :, h, :], vbuf[1, b, :, h, :]])
                s = lax.dot_general(
                    q, k, (((1,), (1,)), ((), ())),
                    preferred_element_type=jnp.float32,
                )
                p = jnp.exp(s)
                l = jnp.sum(p, axis=1, keepdims=True)
                o = lax.dot_general(
                    p.astype(jnp.bfloat16), v, (((1,), (0,)), ((), ())),
                    preferred_element_type=jnp.float32,
                )
                o_ref[b, :, h, :] = o / l

    return pl.pallas_call(
        body,
        out_shape=jax.ShapeDtypeStruct((B, SQ, H, D), jnp.float32),
        in_specs=[pl.BlockSpec(memory_space=pltpu.VMEM)] * 3,
        out_specs=pl.BlockSpec(memory_space=pltpu.VMEM),
        scratch_shapes=[
            pltpu.VMEM((2, B, SQ, H, D), jnp.bfloat16),
            pltpu.VMEM((2, B, SQ, H, D), jnp.bfloat16),
            pltpu.SemaphoreType.DMA((2,)),
            pltpu.SemaphoreType.DMA((2,)),
        ],
        compiler_params=pltpu.CompilerParams(collective_id=0),
    )(Q, K, V)


# device time: 23091 ns/iter; 1.4017x vs baseline; 1.4017x over previous
import jax
import jax.numpy as jnp
from jax import lax
from jax.experimental import pallas as pl
from jax.experimental.pallas import tpu as pltpu

B, SQ, H, D = 2, 256, 8, 64
R = B * SQ
W = H * D
NC = 8
CR = R // NC
NCB = NC // B
SCALE = D ** -0.5


def kernel(Q, K, V):
    Q2 = (Q * SCALE).astype(jnp.bfloat16).reshape(R, W)
    K2 = K.astype(jnp.bfloat16).reshape(R, W)
    V2 = V.astype(jnp.bfloat16).reshape(R, W)

    def body(q_ref, k_ref, v_ref, o_ref, krem, vrem, lbuf,
             y_send, y_recv, f_send, f_recv):
        my_x = lax.axis_index("x")
        my_y = lax.axis_index("y")
        ynbr = (my_x, 1 - my_y)
        xnbr = (1 - my_x, my_y)

        barrier = pltpu.get_barrier_semaphore()
        for peer in (ynbr, xnbr):
            pl.semaphore_signal(
                barrier, inc=1, device_id=peer,
                device_id_type=pl.DeviceIdType.MESH,
            )
        pl.semaphore_wait(barrier, 2)

        def chunk(ref, i):
            return ref.at[pl.ds(CR * i, CR), :]

        def y_desc(src, dst, i):
            return pltpu.make_async_remote_copy(
                src_ref=chunk(src, i), dst_ref=chunk(dst, i),
                send_sem=y_send.at[i], recv_sem=y_recv.at[i],
                device_id=ynbr, device_id_type=pl.DeviceIdType.MESH,
            )

        def f_desc(src, dst, i):
            return pltpu.make_async_remote_copy(
                src_ref=chunk(src, i), dst_ref=chunk(dst, i),
                send_sem=f_send.at[i], recv_sem=f_recv.at[i],
                device_id=xnbr, device_id_type=pl.DeviceIdType.MESH,
            )

        @pl.when(my_x == 0)
        def _():
            for i in range(NC):
                y_desc(k_ref, krem, i).start()

        @pl.when(my_x != 0)
        def _():
            for i in range(NC):
                y_desc(v_ref, vrem, i).start()

        for b in range(B):
            rows = slice(b * SQ, (b + 1) * SQ)
            for h in range(H):
                cols = slice(h * D, (h + 1) * D)
                s = lax.dot_general(
                    q_ref[rows, cols], k_ref[rows, cols],
                    (((1,), (1,)), ((), ())),
                    preferred_element_type=jnp.float32,
                )
                p = jnp.exp(s.astype(jnp.bfloat16))
                lbuf[b * H + h] = jnp.sum(
                    p, axis=1, keepdims=True, dtype=jnp.float32
                )
                o_ref[rows, cols] = lax.dot_general(
                    p, v_ref[rows, cols], (((1,), (0,)), ((), ())),
                    preferred_element_type=jnp.float32,
                )

        for i in range(NC):
            y_desc(k_ref, krem, i).wait_recv()

            @pl.when(my_x == 0)
            def _(i=i):
                f_desc(krem, krem, i).start()

            @pl.when(my_x != 0)
            def _(i=i):
                f_desc(vrem, vrem, i).start()

        for b in range(B):
            for i in range(b * NCB, (b + 1) * NCB):
                f_desc(krem, krem, i).wait_recv()
            rows = slice(b * SQ, (b + 1) * SQ)
            for h in range(H):
                cols = slice(h * D, (h + 1) * D)
                s = lax.dot_general(
                    q_ref[rows, cols], krem[rows, cols],
                    (((1,), (1,)), ((), ())),
                    preferred_element_type=jnp.float32,
                )
                p = jnp.exp(s.astype(jnp.bfloat16))
                l = lbuf[b * H + h] + jnp.sum(
                    p, axis=1, keepdims=True, dtype=jnp.float32
                )
                o = o_ref[rows, cols] + lax.dot_general(
                    p, vrem[rows, cols], (((1,), (0,)), ((), ())),
                    preferred_element_type=jnp.float32,
                )
                o_ref[rows, cols] = o / l

        for i in range(NC):
            y_desc(k_ref, krem, i).wait_send()
            f_desc(krem, krem, i).wait_send()

    out = pl.pallas_call(
        body,
        out_shape=jax.ShapeDtypeStruct((R, W), jnp.float32),
        in_specs=[pl.BlockSpec(memory_space=pltpu.VMEM)] * 3,
        out_specs=pl.BlockSpec(memory_space=pltpu.VMEM),
        scratch_shapes=[
            pltpu.VMEM((R, W), jnp.bfloat16),
            pltpu.VMEM((R, W), jnp.bfloat16),
            pltpu.VMEM((B * H, SQ, 1), jnp.float32),
            pltpu.SemaphoreType.DMA((NC,)),
            pltpu.SemaphoreType.DMA((NC,)),
            pltpu.SemaphoreType.DMA((NC,)),
            pltpu.SemaphoreType.DMA((NC,)),
        ],
        compiler_params=pltpu.CompilerParams(collective_id=0),
    )(Q2, K2, V2)
    return out.reshape(B, SQ, H, D)
